# baseline (speedup 1.0000x reference)
import jax
import jax.numpy as jnp
from jax import lax
from jax.experimental import pallas as pl
from jax.experimental.pallas import tpu as pltpu

_B = 64
_R = 2048
_L = 128
_N = _R * _L
_C = 256
_NC = _R // _C
_Q = _R // 4
_TOPK = 128
_INV_DEN = 1.0 / 511.0
_SCALE = 1.0 / (_B * _TOPK * 2)
_L128 = 0x43000000
_LH_SEED = 0x3F000000
_LN_FULL = 0x48800000
_KEY0 = (0x404CCCCD >> 2) + 0x20000000


def _count_ge(sb_ref, mid):
    qs = [jnp.sum((sb_ref[j * _C:(j + 1) * _C, :] - mid) >> 31)
          for j in range(_NC)]
    while len(qs) > 1:
        qs = [qs[i] + qs[i + 1] for i in range(0, len(qs), 2)]
    return _N + qs[0]


def _init_state(sb, sb_is):
    k0 = jnp.int32(_KEY0)
    ca = jnp.sum((sb[0 * _Q:1 * _Q, :] - k0) >> 31)
    cb = jnp.sum((sb[1 * _Q:2 * _Q, :] - k0) >> 31)
    cd = jnp.sum((sb[2 * _Q:3 * _Q, :] - k0) >> 31)
    ce = jnp.sum((sb[3 * _Q:4 * _Q, :] - k0) >> 31)
    cnt_k0 = _N + ((ca + cb) + (cd + ce))

    lane_max = jnp.max(sb, axis=0)
    lo0 = jnp.min(lane_max)
    hi0 = jnp.max(lane_max) + 1

    k0_ge = cnt_k0 >= _TOPK
    use_k0_lo = k0_ge & (k0 >= lo0)
    lo_i = jnp.where(use_k0_lo, k0, lo0)
    c_lo_i = jnp.where(use_k0_lo, cnt_k0, -1)
    l_cnt_k0 = lax.bitcast_convert_type(
        jnp.maximum(cnt_k0, 1).astype(jnp.float32), jnp.int32)
    l_lo_i = jnp.where(use_k0_lo, l_cnt_k0, jnp.int32(_LN_FULL))
    hi_i = jnp.where(k0_ge, hi0, jnp.minimum(k0, hi0))
    l_hi_i = jnp.where(k0_ge, jnp.int32(_LH_SEED), l_cnt_k0)
    hp_i = jnp.where(k0_ge, jnp.int32(0), jnp.int32(1))
    return (lo_i, hi_i, c_lo_i, l_lo_i, l_hi_i, hp_i)


def _done(st):
    lo, hi, c_lo = st[0], st[1], st[2]
    return (hi - lo <= 1) | (c_lo == _TOPK)


def _propose(st, p):
    lo, hi, _, l_lo, l_hi, hi_probed = st
    b_lo = (lo - jnp.int32(0x20000000)) << 2
    b_hi = (hi - jnp.int32(0x20000000)) << 2
    v_lo = lax.bitcast_convert_type(jnp.maximum(b_lo, 0), jnp.float32)
    v_hi = lax.bitcast_convert_type(jnp.maximum(b_hi, 0), jnp.float32)
    num = (l_lo - _L128).astype(jnp.float32)
    denom = jnp.maximum(l_lo - l_hi, 1).astype(jnp.float32)
    t2_sec = v_lo * v_lo + (num / denom) * (v_hi * v_hi - v_lo * v_lo)
    t2_ext = v_lo * v_lo + jnp.float32(2.0 * 0.6931472 * 2.0 ** -23) * num
    t2 = jnp.where(hi_probed != 0, t2_sec, t2_ext)
    t2b = lax.bitcast_convert_type(t2, jnp.int32)
    r = lax.bitcast_convert_type(jnp.int32(0x5F3759DF) - (t2b >> 1),
                                 jnp.float32)
    r = r * (jnp.float32(1.5) - jnp.float32(0.5) * t2 * r * r)
    r = r * (jnp.float32(1.5) - jnp.float32(0.5) * t2 * r * r)
    v_next = t2 * r
    t_i = (lax.bitcast_convert_type(v_next, jnp.int32) >> 2) \
        + jnp.int32(0x20000000)
    t_i = jnp.where(t2 > 0, t_i, lo)
    t_b = lo + (hi - lo) // 2
    mid = jnp.where(p == 2, t_b, t_i)
    return jnp.clip(mid, lo + 1, hi - 1)


def _update(st, mid, cnt):
    lo, hi, cnt_lo, l_lo, l_hi, hi_probed = st
    l_c = lax.bitcast_convert_type(
        jnp.maximum(cnt, 1).astype(jnp.float32), jnp.int32)
    ge = cnt >= _TOPK
    return (jnp.where(ge, mid, lo), jnp.where(ge, hi, mid),
            jnp.where(ge, cnt, cnt_lo),
            jnp.where(ge, l_c, l_lo), jnp.where(ge, l_hi, l_c),
            jnp.where(ge, hi_probed, jnp.int32(1)))


def _sel_where(st_keep, st_new):
    keep = _done(st_keep)
    return tuple(jnp.where(keep, a, b) for a, b in zip(st_keep, st_new))


def _row_sums(sb_ref, tstar, cnt_ge, p0_ref, p1_ref, q_ref):
    zf = jnp.float32(0.0)

    def part(j0):
        m = sb_ref[j0:j0 + _Q, :] >= tstar
        return (jnp.sum(jnp.where(m, p0_ref[j0:j0 + _Q, :], zf)),
                jnp.sum(jnp.where(m, p1_ref[j0:j0 + _Q, :], zf)),
                jnp.sum(jnp.where(m, q_ref[j0:j0 + _Q, :], zf)))

    pa = part(0 * _Q)
    pb = part(1 * _Q)
    pc = part(2 * _Q)
    pd = part(3 * _Q)
    s0_ge = (pa[0] + pb[0]) + (pc[0] + pd[0])
    s1_ge = (pa[1] + pb[1]) + (pc[1] + pd[1])
    sq_ge = (pa[2] + pb[2]) + (pc[2] + pd[2])

    def no_ties(_):
        return s0_ge, s1_ge, sq_ge

    def with_ties(_):
        sbv = sb_ref[...]
        m_eq = sbv == tstar
        cnt_eq = jnp.sum(m_eq.astype(jnp.int32))
        r = _TOPK - (cnt_ge - cnt_eq)
        rr = lax.broadcasted_iota(jnp.int32, (_R, _L), 0)
        cc = lax.broadcasted_iota(jnp.int32, (_R, _L), 1)
        k = rr * _L + cc

        def j_cond(c):
            lo_j, hi_j = c
            return hi_j - lo_j > 1

        def j_body(c):
            lo_j, hi_j = c
            mid = lo_j + (hi_j - lo_j) // 2
            c_eq = jnp.sum((m_eq & (k < mid)).astype(jnp.int32))
            ge = c_eq >= r
            return (jnp.where(ge, lo_j, mid), jnp.where(ge, mid, hi_j))

        _, cut = lax.while_loop(j_cond, j_body,
                                (jnp.int32(0), jnp.int32(_N)))
        m_sel = (sbv > tstar) | (m_eq & (k < cut))
        return (jnp.sum(jnp.where(m_sel, p0_ref[...], zf)),
                jnp.sum(jnp.where(m_sel, p1_ref[...], zf)),
                jnp.sum(jnp.where(m_sel, q_ref[...], zf)))

    return lax.cond(cnt_ge == _TOPK, no_ties, with_ties, 0)


_NROWS = 16


def _pair_body(x_ref, cr_ref, out_ref, sb0_ref, sb1_ref, sb2_ref, sb3_ref,
               sb4_ref, sb5_ref, sb6_ref, sb7_ref, sb8_ref, sb9_ref,
               sb10_ref, sb11_ref, sb12_ref, sb13_ref, sb14_ref, sb15_ref,
               p0_ref, p1_ref, q_ref):
    i = pl.program_id(0)
    sb_refs = (sb0_ref, sb1_ref, sb2_ref, sb3_ref,
               sb4_ref, sb5_ref, sb6_ref, sb7_ref,
               sb8_ref, sb9_ref, sb10_ref, sb11_ref,
               sb12_ref, sb13_ref, sb14_ref, sb15_ref)

    @pl.when(i == 0)
    def _coords():
        rr = lax.broadcasted_iota(jnp.int32, (_R, _L), 0)
        cc = lax.broadcasted_iota(jnp.int32, (_R, _L), 1)
        p0 = (rr >> 2).astype(jnp.float32) * _INV_DEN
        p1 = ((rr & 3) * _L + cc).astype(jnp.float32) * _INV_DEN
        p0_ref[...] = p0
        p1_ref[...] = p1
        q_ref[...] = 0.5 * (p0 * p0 + p1 * p1)

    def transform(x):
        xb = lax.bitcast_convert_type(x, jnp.int32)
        key = xb ^ ((xb >> 31) & jnp.int32(0x7FFFFFFF))
        return (key >> 2) + jnp.int32(0x20000000)

    states = []
    for rix in range(_NROWS):
        sbv = transform(x_ref[rix])
        sb_refs[rix][...] = sbv
        states.append(_init_state(sbv, sb_refs[rix]))

    def bs_cond(c):
        done = _done(c[0:6])
        for rix in range(1, _NROWS):
            done = done & _done(c[6 * rix:6 * rix + 6])
        return ~done

    def bs_body(c):
        p = c[6 * _NROWS]
        mids = [_propose(c[6 * rix:6 * rix + 6], p)
                for rix in range(_NROWS)]
        cnts = [_count_ge(sb_refs[rix], mids[rix])
                for rix in range(_NROWS)]
        out = ()
        for rix in range(_NROWS):
            st = c[6 * rix:6 * rix + 6]
            out = out + _sel_where(st, _update(st, mids[rix], cnts[rix]))
        return out + (jnp.where(p == 2, 0, p + 1),)

    carry0 = ()
    for st in states:
        carry0 = carry0 + st
    fin = lax.while_loop(bs_cond, bs_body, carry0 + (jnp.int32(0),))

    contrib = jnp.zeros((1, 1), jnp.float32)
    for rix in range(_NROWS):
        t_r = fin[6 * rix]
        c_r = fin[6 * rix + 2]
        c_r = lax.cond(c_r < 0,
                       lambda _, rix=rix, t_r=t_r:
                       _count_ge(sb_refs[rix], t_r),
                       lambda _: c_r, 0)
        s0, s1, sq = _row_sums(sb_refs[rix], t_r, c_r,
                               p0_ref, p1_ref, q_ref)
        c0 = cr_ref[rix, 0:1, :]
        c1 = cr_ref[rix, 1:2, :]
        contrib = contrib + (sq + (_TOPK * 0.5) * (c0 * c0 + c1 * c1)
                             - c0 * s0 - c1 * s1)
    contrib = contrib * _SCALE

    @pl.when(i == 0)
    def _init():
        out_ref[...] = jnp.zeros_like(out_ref)

    out_ref[...] += contrib


def kernel(cls_input, center_rate):
    x3 = cls_input.reshape(_B, _R, _L)
    cr3 = center_rate.T.reshape(_B, 2, 1)
    out = pl.pallas_call(
        _pair_body,
        grid=(_B // _NROWS,),
        in_specs=[
            pl.BlockSpec((_NROWS, _R, _L), lambda i: (i, 0, 0)),
            pl.BlockSpec((_NROWS, 2, 1), lambda i: (i, 0, 0)),
        ],
        out_specs=pl.BlockSpec((1, 1), lambda i: (0, 0)),
        out_shape=jax.ShapeDtypeStruct((1, 1), jnp.float32),
        scratch_shapes=[
            pltpu.VMEM((_R, _L), jnp.int32),
            pltpu.VMEM((_R, _L), jnp.int32),
            pltpu.VMEM((_R, _L), jnp.int32),
            pltpu.VMEM((_R, _L), jnp.int32),
            pltpu.VMEM((_R, _L), jnp.int32),
            pltpu.VMEM((_R, _L), jnp.int32),
            pltpu.VMEM((_R, _L), jnp.int32),
            pltpu.VMEM((_R, _L), jnp.int32),
            pltpu.VMEM((_R, _L), jnp.int32),
            pltpu.VMEM((_R, _L), jnp.int32),
            pltpu.VMEM((_R, _L), jnp.int32),
            pltpu.VMEM((_R, _L), jnp.int32),
            pltpu.VMEM((_R, _L), jnp.int32),
            pltpu.VMEM((_R, _L), jnp.int32),
            pltpu.VMEM((_R, _L), jnp.int32),
            pltpu.VMEM((_R, _L), jnp.int32),
            pltpu.VMEM((_R, _L), jnp.float32),
            pltpu.VMEM((_R, _L), jnp.float32),
            pltpu.VMEM((_R, _L), jnp.float32),
        ],
    )(x3, cr3)
    return out[0, 0]


# speedup vs baseline: 1.3323x; 1.3323x over previous
import jax
import jax.numpy as jnp
from jax import lax
from jax.experimental import pallas as pl
from jax.experimental.pallas import tpu as pltpu

_B = 64
_R = 2048
_L = 128
_N = _R * _L
_C = 256
_NC = _R // _C
_Q = _R // 4
_TOPK = 128
_INV_DEN = 1.0 / 511.0
_SCALE = 1.0 / (_B * _TOPK * 2)
_L128 = 0x43000000
_LH_SEED = 0x3F000000
_LN_FULL = 0x48800000
_KEY0 = (0x404CCCCD >> 2) + 0x20000000


def _count_ge(sb_ref, mid):
    qs = [jnp.sum((sb_ref[j * _C:(j + 1) * _C, :] - mid) >> 31)
          for j in range(_NC)]
    while len(qs) > 1:
        qs = [qs[i] + qs[i + 1] for i in range(0, len(qs), 2)]
    return _N + qs[0]


def _init_state(sb, sb_is):
    k0 = jnp.int32(_KEY0)
    ca = jnp.sum((sb[0 * _Q:1 * _Q, :] - k0) >> 31)
    cb = jnp.sum((sb[1 * _Q:2 * _Q, :] - k0) >> 31)
    cd = jnp.sum((sb[2 * _Q:3 * _Q, :] - k0) >> 31)
    ce = jnp.sum((sb[3 * _Q:4 * _Q, :] - k0) >> 31)
    cnt_k0 = _N + ((ca + cb) + (cd + ce))

    lane_max = jnp.max(sb, axis=0)
    lo0 = jnp.min(lane_max)
    hi0 = jnp.max(lane_max) + 1

    k0_ge = cnt_k0 >= _TOPK
    use_k0_lo = k0_ge & (k0 >= lo0)
    lo_i = jnp.where(use_k0_lo, k0, lo0)
    c_lo_i = jnp.where(use_k0_lo, cnt_k0, -1)
    l_cnt_k0 = lax.bitcast_convert_type(
        jnp.maximum(cnt_k0, 1).astype(jnp.float32), jnp.int32)
    l_lo_i = jnp.where(use_k0_lo, l_cnt_k0, jnp.int32(_LN_FULL))
    hi_i = jnp.where(k0_ge, hi0, jnp.minimum(k0, hi0))
    l_hi_i = jnp.where(k0_ge, jnp.int32(_LH_SEED), l_cnt_k0)
    hp_i = jnp.where(k0_ge, jnp.int32(0), jnp.int32(1))
    return (lo_i, hi_i, c_lo_i, l_lo_i, l_hi_i, hp_i)


def _done(st):
    lo, hi, c_lo = st[0], st[1], st[2]
    return (hi - lo <= 1) | (c_lo == _TOPK)


def _propose(st, p):
    lo, hi, _, l_lo, l_hi, hi_probed = st
    b_lo = (lo - jnp.int32(0x20000000)) << 2
    b_hi = (hi - jnp.int32(0x20000000)) << 2
    v_lo = lax.bitcast_convert_type(jnp.maximum(b_lo, 0), jnp.float32)
    v_hi = lax.bitcast_convert_type(jnp.maximum(b_hi, 0), jnp.float32)
    num = (l_lo - _L128).astype(jnp.float32)
    denom = jnp.maximum(l_lo - l_hi, 1).astype(jnp.float32)
    t2_sec = v_lo * v_lo + (num / denom) * (v_hi * v_hi - v_lo * v_lo)
    t2_ext = v_lo * v_lo + jnp.float32(2.0 * 0.6931472 * 2.0 ** -23) * num
    t2 = jnp.where(hi_probed != 0, t2_sec, t2_ext)
    t2b = lax.bitcast_convert_type(t2, jnp.int32)
    r = lax.bitcast_convert_type(jnp.int32(0x5F3759DF) - (t2b >> 1),
                                 jnp.float32)
    r = r * (jnp.float32(1.5) - jnp.float32(0.5) * t2 * r * r)
    r = r * (jnp.float32(1.5) - jnp.float32(0.5) * t2 * r * r)
    v_next = t2 * r
    t_i = (lax.bitcast_convert_type(v_next, jnp.int32) >> 2) \
        + jnp.int32(0x20000000)
    t_i = jnp.where(t2 > 0, t_i, lo)
    t_b = lo + (hi - lo) // 2
    mid = jnp.where(p == 2, t_b, t_i)
    return jnp.clip(mid, lo + 1, hi - 1)


def _update(st, mid, cnt):
    lo, hi, cnt_lo, l_lo, l_hi, hi_probed = st
    l_c = lax.bitcast_convert_type(
        jnp.maximum(cnt, 1).astype(jnp.float32), jnp.int32)
    ge = cnt >= _TOPK
    return (jnp.where(ge, mid, lo), jnp.where(ge, hi, mid),
            jnp.where(ge, cnt, cnt_lo),
            jnp.where(ge, l_c, l_lo), jnp.where(ge, l_hi, l_c),
            jnp.where(ge, hi_probed, jnp.int32(1)))


def _sel_where(st_keep, st_new):
    keep = _done(st_keep)
    return tuple(jnp.where(keep, a, b) for a, b in zip(st_keep, st_new))


def _row_sums(sb_ref, tstar, cnt_ge, p0_ref, p1_ref, q_ref):
    zf = jnp.float32(0.0)

    def part(j0):
        m = sb_ref[j0:j0 + _Q, :] >= tstar
        return (jnp.sum(jnp.where(m, p0_ref[j0:j0 + _Q, :], zf)),
                jnp.sum(jnp.where(m, p1_ref[j0:j0 + _Q, :], zf)),
                jnp.sum(jnp.where(m, q_ref[j0:j0 + _Q, :], zf)))

    pa = part(0 * _Q)
    pb = part(1 * _Q)
    pc = part(2 * _Q)
    pd = part(3 * _Q)
    s0_ge = (pa[0] + pb[0]) + (pc[0] + pd[0])
    s1_ge = (pa[1] + pb[1]) + (pc[1] + pd[1])
    sq_ge = (pa[2] + pb[2]) + (pc[2] + pd[2])

    def no_ties(_):
        return s0_ge, s1_ge, sq_ge

    def with_ties(_):
        sbv = sb_ref[...]
        m_eq = sbv == tstar
        cnt_eq = jnp.sum(m_eq.astype(jnp.int32))
        r = _TOPK - (cnt_ge - cnt_eq)
        rr = lax.broadcasted_iota(jnp.int32, (_R, _L), 0)
        cc = lax.broadcasted_iota(jnp.int32, (_R, _L), 1)
        k = rr * _L + cc

        def j_cond(c):
            lo_j, hi_j = c
            return hi_j - lo_j > 1

        def j_body(c):
            lo_j, hi_j = c
            mid = lo_j + (hi_j - lo_j) // 2
            c_eq = jnp.sum((m_eq & (k < mid)).astype(jnp.int32))
            ge = c_eq >= r
            return (jnp.where(ge, lo_j, mid), jnp.where(ge, mid, hi_j))

        _, cut = lax.while_loop(j_cond, j_body,
                                (jnp.int32(0), jnp.int32(_N)))
        m_sel = (sbv > tstar) | (m_eq & (k < cut))
        return (jnp.sum(jnp.where(m_sel, p0_ref[...], zf)),
                jnp.sum(jnp.where(m_sel, p1_ref[...], zf)),
                jnp.sum(jnp.where(m_sel, q_ref[...], zf)))

    return lax.cond(cnt_ge == _TOPK, no_ties, with_ties, 0)


_NROWS = 8


def _pair_body(x_ref, cr_ref, out_ref, sb0_ref, sb1_ref, sb2_ref, sb3_ref,
               sb4_ref, sb5_ref, sb6_ref, sb7_ref, p0_ref, p1_ref, q_ref):
    i = pl.program_id(0)
    sb_refs = (sb0_ref, sb1_ref, sb2_ref, sb3_ref,
               sb4_ref, sb5_ref, sb6_ref, sb7_ref)

    @pl.when(i == 0)
    def _coords():
        rr = lax.broadcasted_iota(jnp.int32, (_R, _L), 0)
        cc = lax.broadcasted_iota(jnp.int32, (_R, _L), 1)
        p0 = (rr >> 2).astype(jnp.float32) * _INV_DEN
        p1 = ((rr & 3) * _L + cc).astype(jnp.float32) * _INV_DEN
        p0_ref[...] = p0
        p1_ref[...] = p1
        q_ref[...] = 0.5 * (p0 * p0 + p1 * p1)

    def transform(x):
        xb = lax.bitcast_convert_type(x, jnp.int32)
        key = xb ^ ((xb >> 31) & jnp.int32(0x7FFFFFFF))
        return (key >> 2) + jnp.int32(0x20000000)

    states = []
    for rix in range(_NROWS):
        sbv = transform(x_ref[rix])
        sb_refs[rix][...] = sbv
        states.append(_init_state(sbv, sb_refs[rix]))

    def bs_cond(c):
        done = _done(c[0:6])
        for rix in range(1, _NROWS):
            done = done & _done(c[6 * rix:6 * rix + 6])
        return ~done

    def bs_body(c):
        p = c[6 * _NROWS]
        mids = [_propose(c[6 * rix:6 * rix + 6], p)
                for rix in range(_NROWS)]
        cnts = [_count_ge(sb_refs[rix], mids[rix])
                for rix in range(_NROWS)]
        out = ()
        for rix in range(_NROWS):
            st = c[6 * rix:6 * rix + 6]
            out = out + _sel_where(st, _update(st, mids[rix], cnts[rix]))
        return out + (jnp.where(p == 2, 0, p + 1),)

    carry0 = ()
    for st in states:
        carry0 = carry0 + st
    fin = lax.while_loop(bs_cond, bs_body, carry0 + (jnp.int32(0),))

    contrib = jnp.zeros((1, 1), jnp.float32)
    for rix in range(_NROWS):
        t_r = fin[6 * rix]
        c_r = fin[6 * rix + 2]
        c_r = lax.cond(c_r < 0,
                       lambda _, rix=rix, t_r=t_r:
                       _count_ge(sb_refs[rix], t_r),
                       lambda _: c_r, 0)
        s0, s1, sq = _row_sums(sb_refs[rix], t_r, c_r,
                               p0_ref, p1_ref, q_ref)
        c0 = cr_ref[rix, 0:1, :]
        c1 = cr_ref[rix, 1:2, :]
        contrib = contrib + (sq + (_TOPK * 0.5) * (c0 * c0 + c1 * c1)
                             - c0 * s0 - c1 * s1)
    contrib = contrib * _SCALE

    @pl.when(i == 0)
    def _init():
        out_ref[...] = jnp.zeros_like(out_ref)

    out_ref[...] += contrib


def kernel(cls_input, center_rate):
    x3 = cls_input.reshape(_B, _R, _L)
    cr3 = center_rate.T.reshape(_B, 2, 1)
    out = pl.pallas_call(
        _pair_body,
        grid=(_B // _NROWS,),
        in_specs=[
            pl.BlockSpec((_NROWS, _R, _L), lambda i: (i, 0, 0)),
            pl.BlockSpec((_NROWS, 2, 1), lambda i: (i, 0, 0)),
        ],
        out_specs=pl.BlockSpec((1, 1), lambda i: (0, 0)),
        out_shape=jax.ShapeDtypeStruct((1, 1), jnp.float32),
        scratch_shapes=[
            pltpu.VMEM((_R, _L), jnp.int32),
            pltpu.VMEM((_R, _L), jnp.int32),
            pltpu.VMEM((_R, _L), jnp.int32),
            pltpu.VMEM((_R, _L), jnp.int32),
            pltpu.VMEM((_R, _L), jnp.int32),
            pltpu.VMEM((_R, _L), jnp.int32),
            pltpu.VMEM((_R, _L), jnp.int32),
            pltpu.VMEM((_R, _L), jnp.int32),
            pltpu.VMEM((_R, _L), jnp.float32),
            pltpu.VMEM((_R, _L), jnp.float32),
            pltpu.VMEM((_R, _L), jnp.float32),
        ],
    )(x3, cr3)
    return out[0, 0]
